# baseline (device time: 16641 ns/iter reference)
import jax
import jax.numpy as jnp
from jax import lax
from jax.experimental import pallas as pl
from jax.experimental.pallas import tpu as pltpu

N_Z = 2
C = 4


def kernel(x):
    _, m, n_total = x.shape
    half = n_total // N_Z
    qr = m // 4
    cr = qr // C

    def body(
        x_ref,
        out_ref,
        local_ref,
        recv_z_ref,
        sem_local_cp,
        sem_out,
        sem_sz,
        sem_rz,
        sem_sx,
        sem_rx,
        sem_sy,
        sem_ry,
        sem_sd,
        sem_rd,
    ):
        my_x = lax.axis_index("x")
        my_y = lax.axis_index("y")
        my_z = lax.axis_index("z")
        other_z = 1 - my_z
        other_x = 1 - my_x
        other_y = 1 - my_y
        r_me = (2 * my_x + my_y) * qr
        r_xp = (2 * other_x + my_y) * qr
        r_yp = (2 * my_x + other_y) * qr
        r_dg = (2 * other_x + other_y) * qr

        local_cp = pltpu.make_async_copy(
            x_ref.at[0, pl.ds(r_me, qr), pl.ds(my_z * half, half)],
            local_ref,
            sem_local_cp,
        )
        local_cp.start()

        barrier_sem = pltpu.get_barrier_semaphore()
        for dev in (
            (my_x, my_y, other_z),
            (other_x, my_y, my_z),
            (my_x, other_y, my_z),
            (other_x, other_y, my_z),
        ):
            pl.semaphore_signal(
                barrier_sem, inc=1,
                device_id=dev,
                device_id_type=pl.DeviceIdType.MESH,
            )
        pl.semaphore_wait(barrier_sem, 4)

        rdma_z = []
        for i in range(C):
            d = pltpu.make_async_remote_copy(
                src_ref=x_ref.at[
                    0, pl.ds(r_me + i * cr, cr), pl.ds(other_z * half, half)
                ],
                dst_ref=recv_z_ref.at[i],
                send_sem=sem_sz.at[i],
                recv_sem=sem_rz.at[i],
                device_id=(my_x, my_y, other_z),
                device_id_type=pl.DeviceIdType.MESH,
            )
            d.start()
            rdma_z.append(d)

        local_cp.wait()

        gather = []
        out_cp = []
        for i in range(C):
            rdma_z[i].wait_recv()
            local_ref[pl.ds(i * cr, cr), :] = (
                local_ref[pl.ds(i * cr, cr), :] + recv_z_ref[i, :, :]
            )
            for dev, ss, rs in (
                ((other_x, my_y, my_z), sem_sx, sem_rx),
                ((my_x, other_y, my_z), sem_sy, sem_ry),
                ((other_x, other_y, my_z), sem_sd, sem_rd),
            ):
                d = pltpu.make_async_remote_copy(
                    src_ref=local_ref.at[pl.ds(i * cr, cr), :],
                    dst_ref=out_ref.at[pl.ds(r_me + i * cr, cr), :],
                    send_sem=ss.at[i],
                    recv_sem=rs.at[i],
                    device_id=dev,
                    device_id_type=pl.DeviceIdType.MESH,
                )
                d.start()
                gather.append(d)
            cp = pltpu.make_async_copy(
                local_ref.at[pl.ds(i * cr, cr), :],
                out_ref.at[pl.ds(r_me + i * cr, cr), :],
                sem_out.at[i],
            )
            cp.start()
            out_cp.append(cp)

        for i in range(C):
            rdma_z[i].wait_send()
            for g in gather[3 * i : 3 * i + 3]:
                g.wait_send()
            out_cp[i].wait()
        for r_peer, rs in ((r_xp, sem_rx), (r_yp, sem_ry), (r_dg, sem_rd)):
            for i in range(C):
                recv = pltpu.make_async_remote_copy(
                    src_ref=local_ref.at[pl.ds(i * cr, cr), :],
                    dst_ref=out_ref.at[pl.ds(r_peer + i * cr, cr), :],
                    send_sem=sem_sz.at[i],
                    recv_sem=rs.at[i],
                    device_id=(my_x, my_y, my_z),
                    device_id_type=pl.DeviceIdType.MESH,
                )
                recv.wait_recv()

    return pl.pallas_call(
        body,
        out_shape=jax.ShapeDtypeStruct((m, half), jnp.float32),
        in_specs=[pl.BlockSpec(memory_space=pl.ANY)],
        out_specs=pl.BlockSpec(memory_space=pl.ANY),
        scratch_shapes=[
            pltpu.VMEM((qr, half), jnp.float32),
            pltpu.VMEM((C, cr, half), jnp.float32),
            pltpu.SemaphoreType.DMA,
        ]
        + [pltpu.SemaphoreType.DMA((C,))] * 9,
        compiler_params=pltpu.CompilerParams(collective_id=0),
    )(x)


# device time: 15698 ns/iter; 1.0601x vs baseline; 1.0601x over previous
import jax
import jax.numpy as jnp
from jax import lax
from jax.experimental import pallas as pl
from jax.experimental.pallas import tpu as pltpu

N_Z = 2
C = 8


def kernel(x):
    _, m, n_total = x.shape
    half = n_total // N_Z
    rows = m // 2
    cr = rows // C

    def body(
        x_ref,
        out_ref,
        local_ref,
        recv_z_ref,
        sem_local_cp,
        sem_out,
        sem_sz,
        sem_rz,
        sem_sx,
        sem_rx,
    ):
        my_x = lax.axis_index("x")
        my_y = lax.axis_index("y")
        my_z = lax.axis_index("z")
        other_z = 1 - my_z
        other_x = 1 - my_x
        r0 = my_x * rows

        local_cp = pltpu.make_async_copy(
            x_ref.at[0, pl.ds(r0, rows), pl.ds(my_z * half, half)],
            local_ref,
            sem_local_cp,
        )
        local_cp.start()

        barrier_sem = pltpu.get_barrier_semaphore()
        pl.semaphore_signal(
            barrier_sem, inc=1,
            device_id=(my_x, my_y, other_z),
            device_id_type=pl.DeviceIdType.MESH,
        )
        pl.semaphore_signal(
            barrier_sem, inc=1,
            device_id=(other_x, my_y, my_z),
            device_id_type=pl.DeviceIdType.MESH,
        )
        pl.semaphore_wait(barrier_sem, 2)

        rdma_z = []
        for i in range(C):
            d = pltpu.make_async_remote_copy(
                src_ref=x_ref.at[
                    0, pl.ds(r0 + i * cr, cr), pl.ds(other_z * half, half)
                ],
                dst_ref=recv_z_ref.at[i],
                send_sem=sem_sz.at[i],
                recv_sem=sem_rz.at[i],
                device_id=(my_x, my_y, other_z),
                device_id_type=pl.DeviceIdType.MESH,
            )
            d.start()
            rdma_z.append(d)

        local_cp.wait()

        rdma_x = []
        out_cp = []
        for i in range(C):
            rdma_z[i].wait_recv()
            local_ref[pl.ds(i * cr, cr), :] = (
                local_ref[pl.ds(i * cr, cr), :] + recv_z_ref[i, :, :]
            )
            d = pltpu.make_async_remote_copy(
                src_ref=local_ref.at[pl.ds(i * cr, cr), :],
                dst_ref=out_ref.at[pl.ds(r0 + i * cr, cr), :],
                send_sem=sem_sx.at[i],
                recv_sem=sem_rx.at[i],
                device_id=(other_x, my_y, my_z),
                device_id_type=pl.DeviceIdType.MESH,
            )
            d.start()
            rdma_x.append(d)
            cp = pltpu.make_async_copy(
                local_ref.at[pl.ds(i * cr, cr), :],
                out_ref.at[pl.ds(r0 + i * cr, cr), :],
                sem_out.at[i],
            )
            cp.start()
            out_cp.append(cp)

        for i in range(C):
            rdma_z[i].wait_send()
            rdma_x[i].wait()
            out_cp[i].wait()

    return pl.pallas_call(
        body,
        out_shape=jax.ShapeDtypeStruct((m, half), jnp.float32),
        in_specs=[pl.BlockSpec(memory_space=pl.ANY)],
        out_specs=pl.BlockSpec(memory_space=pl.ANY),
        scratch_shapes=[
            pltpu.VMEM((rows, half), jnp.float32),
            pltpu.VMEM((C, cr, half), jnp.float32),
            pltpu.SemaphoreType.DMA,
            pltpu.SemaphoreType.DMA((C,)),
            pltpu.SemaphoreType.DMA((C,)),
            pltpu.SemaphoreType.DMA((C,)),
            pltpu.SemaphoreType.DMA((C,)),
            pltpu.SemaphoreType.DMA((C,)),
        ],
        compiler_params=pltpu.CompilerParams(collective_id=0),
    )(x)
